# Initial kernel scaffold; baseline (speedup 1.0000x reference)
import jax
import jax.numpy as jnp
from jax.experimental import pallas as pl


def kernel(x, edge_index, W_lin, att, W_out, b_out):
    raise NotImplementedError("write your pallas kernel here")


# speedup vs baseline: 7.7256x; 7.7256x over previous
import functools

import jax
import jax.numpy as jnp
from jax import lax
from jax.experimental import pallas as pl
from jax.experimental.pallas import tpu as pltpu
from jax.experimental.pallas import tpu_sc as plsc

N_NODES = 10000
N_EDGES = 320000
HEADS = 8
OUT_CH = 32
NEG = 0.2

NW = 32
A_CHUNK = 2000
A_EPT = N_EDGES // 4
B_CHUNK = 80
B_EPT = N_EDGES // NW


def _tc_dense(x, W_lin, B, attW):
    def body(x_ref, wl_ref, b_ref, aw_ref, y_ref, at_ref):
        xl = lax.dot_general(x_ref[...], wl_ref[...], (((1,), (1,)), ((), ())),
                             preferred_element_type=jnp.float32)
        y_ref[...] = jnp.dot(xl, b_ref[...], preferred_element_type=jnp.float32)
        at_ref[...] = lax.dot_general(
            aw_ref[...], xl, (((0,), (1,)), ((), ())),
            preferred_element_type=jnp.float32)

    return pl.pallas_call(
        body,
        out_shape=(
            jax.ShapeDtypeStruct((N_NODES, 256), jnp.float32),
            jax.ShapeDtypeStruct((16, N_NODES), jnp.float32),
        ),
    )(x, W_lin, B, attW)


def _lrelu(v):
    return jnp.where(v >= 0.0, v, NEG * v)


def _pass_a(AT, row, col):
    mesh = plsc.VectorSubcoreMesh(core_axis_name="c", subcore_axis_name="s")

    @functools.partial(
        pl.kernel, mesh=mesh,
        compiler_params=pltpu.CompilerParams(needs_layout_passes=False),
        out_type=(
            jax.ShapeDtypeStruct((HEADS * N_EDGES,), jnp.float32),
            jax.ShapeDtypeStruct((NW * 16,), jnp.float32),
        ),
        scratch_types=[
            pltpu.VMEM((N_NODES,), jnp.float32),
            pltpu.VMEM((N_NODES,), jnp.float32),
            pltpu.VMEM((A_CHUNK,), jnp.int32),
            pltpu.VMEM((A_CHUNK,), jnp.int32),
            pltpu.VMEM((A_CHUNK,), jnp.float32),
            pltpu.VMEM((16,), jnp.float32),
        ],
    )
    def k(at_hbm, row_hbm, col_hbm, p_hbm, part_hbm, al_v, ar_v, ir_v, ic_v,
          po_v, sp_v):
        wid = lax.axis_index("s") * 2 + lax.axis_index("c")
        h = wid // 4
        q = wid % 4
        base = q * A_EPT

        pltpu.sync_copy(at_hbm.at[pl.ds(h * N_NODES, N_NODES)], al_v)
        pltpu.sync_copy(at_hbm.at[pl.ds((h + HEADS) * N_NODES, N_NODES)], ar_v)

        def mx(i, carry):
            ml, mr = carry
            ml = jnp.maximum(ml, al_v[pl.ds(i * 16, 16)])
            mr = jnp.maximum(mr, ar_v[pl.ds(i * 16, 16)])
            return ml, mr
        neg = jnp.full((16,), -3e38, jnp.float32)
        ml, mr = lax.fori_loop(0, N_NODES // 16, mx, (neg, neg))

        lanes = lax.iota(jnp.int32, 16)

        def butterfly(v, op):
            for k in (1, 2, 4, 8):
                sp_v[...] = v
                v = op(v, plsc.load_gather(sp_v, [lanes ^ k]))
            return v

        shift = _lrelu(butterfly(ml, jnp.maximum) + butterfly(mr, jnp.maximum))

        def chunk(cidx, acc):
            ebase = base + cidx * A_CHUNK
            pltpu.sync_copy(row_hbm.at[pl.ds(ebase, A_CHUNK)], ir_v)
            pltpu.sync_copy(col_hbm.at[pl.ds(ebase, A_CHUNK)], ic_v)

            def grp(g, acc):
                rv = ir_v[pl.ds(g * 16, 16)]
                cv = ic_v[pl.ds(g * 16, 16)]
                s = plsc.load_gather(al_v, [rv]) + plsc.load_gather(ar_v, [cv])
                p = jnp.exp(_lrelu(s) - shift)
                po_v[pl.ds(g * 16, 16)] = p
                return acc + p

            acc = lax.fori_loop(0, A_CHUNK // 16, grp, acc)
            pltpu.sync_copy(po_v, p_hbm.at[pl.ds(h * N_EDGES + ebase, A_CHUNK)])
            return acc

        acc = lax.fori_loop(0, A_EPT // A_CHUNK, chunk, jnp.zeros((16,), jnp.float32))
        total = butterfly(acc, jnp.add)
        sp_v[...] = total
        pltpu.sync_copy(sp_v, part_hbm.at[pl.ds(wid * 16, 16)])

    return k(AT, row, col)


def _pass_b(Y, col, p, part, b_out):
    mesh = plsc.VectorSubcoreMesh(core_axis_name="c", subcore_axis_name="s")

    @functools.partial(
        pl.kernel, mesh=mesh,
        compiler_params=pltpu.CompilerParams(needs_layout_passes=False),
        out_type=jax.ShapeDtypeStruct((N_EDGES, OUT_CH), jnp.float32),
        scratch_types=[
            pltpu.VMEM((B_CHUNK,), jnp.int32),
            pltpu.VMEM((B_CHUNK, 256), jnp.float32),
            pltpu.VMEM((HEADS, B_CHUNK), jnp.float32),
            pltpu.VMEM((B_CHUNK, OUT_CH), jnp.float32),
            pltpu.VMEM((NW * 16,), jnp.float32),
            pltpu.VMEM((HEADS, 16), jnp.float32),
            pltpu.VMEM((OUT_CH,), jnp.float32),
            pltpu.SemaphoreType.DMA,
        ],
    )
    def k(y_hbm, col_hbm, p_hbm, part_hbm, b_hbm, out_hbm, ic_v, y_v, p_v,
          o_v, part_v, inv_v, b_v, sem):
        wid = lax.axis_index("s") * 2 + lax.axis_index("c")
        base = wid * B_EPT

        pltpu.sync_copy(part_hbm, part_v)
        pltpu.sync_copy(b_hbm, b_v)
        for h in range(HEADS):
            d = (part_v[pl.ds((4 * h) * 16, 16)]
                 + part_v[pl.ds((4 * h + 1) * 16, 16)]
                 + part_v[pl.ds((4 * h + 2) * 16, 16)]
                 + part_v[pl.ds((4 * h + 3) * 16, 16)])
            inv_v[h] = 1.0 / d
        blo = b_v[pl.ds(0, 16)]
        bhi = b_v[pl.ds(16, 16)]

        def chunk(cidx, _):
            ebase = base + cidx * B_CHUNK
            pltpu.sync_copy(col_hbm.at[pl.ds(ebase, B_CHUNK)], ic_v)
            gath = pltpu.async_copy(y_hbm.at[ic_v], y_v, sem)
            for h in range(HEADS):
                pltpu.sync_copy(p_hbm.at[pl.ds(h * N_EDGES + ebase, B_CHUNK)],
                                p_v.at[h])
            for h in range(HEADS):
                iv = inv_v[h]
                for g in range(B_CHUNK // 16):
                    p_v[h, pl.ds(g * 16, 16)] = p_v[h, pl.ds(g * 16, 16)] * iv
            gath.wait()

            def edge(e, _):
                acc_lo = blo
                acc_hi = bhi
                for h in range(HEADS):
                    a = plsc.load_gather(
                        p_v, [jnp.full((16,), h, jnp.int32),
                              jnp.full((16,), 1, jnp.int32) * e])
                    acc_lo = acc_lo + a * y_v[e, pl.ds(h * 32, 16)]
                    acc_hi = acc_hi + a * y_v[e, pl.ds(h * 32 + 16, 16)]
                o_v[e, pl.ds(0, 16)] = acc_lo
                o_v[e, pl.ds(16, 16)] = acc_hi
                return 0

            lax.fori_loop(0, B_CHUNK, edge, 0)
            pltpu.sync_copy(o_v, out_hbm.at[pl.ds(ebase, B_CHUNK)])
            return 0

        lax.fori_loop(0, B_EPT // B_CHUNK, chunk, 0)

    return k(Y, col, p, part, b_out)


def kernel(x, edge_index, W_lin, att, W_out, b_out):
    row = edge_index[0].astype(jnp.int32)
    col = edge_index[1].astype(jnp.int32)

    eye8 = jnp.eye(HEADS, dtype=jnp.float32)
    W_t = W_out.reshape(OUT_CH, HEADS, OUT_CH).transpose(1, 2, 0)
    B = (eye8[:, None, :, None] * W_t[:, :, None, :]).reshape(256, 256)
    att_l = att[0, :, :OUT_CH]
    att_r = att[0, :, OUT_CH:]
    attW_l = (eye8[:, None, :] * att_l[:, :, None]).reshape(256, HEADS)
    attW_r = (eye8[:, None, :] * att_r[:, :, None]).reshape(256, HEADS)
    attW = jnp.concatenate([attW_l, attW_r], axis=1)

    Y, AT = _tc_dense(x, W_lin, B, attW)
    p, part = _pass_a(AT.reshape(-1), row, col)
    out = _pass_b(Y, col, p, part, b_out)
    return out


# speedup vs baseline: 15.8388x; 2.0502x over previous
import functools

import jax
import jax.numpy as jnp
from jax import lax
from jax.experimental import pallas as pl
from jax.experimental.pallas import tpu as pltpu
from jax.experimental.pallas import tpu_sc as plsc

N_NODES = 10000
N_EDGES = 320000
HEADS = 8
OUT_CH = 32
NEG = 0.2

NW = 32
A_CHUNK = 2000
A_EPT = N_EDGES // 4
B_CHUNK = 80
B_EPT = N_EDGES // NW


def _tc_dense(x, W_lin, B, attW):
    def body(x_ref, wl_ref, b_ref, aw_ref, y_ref, at_ref):
        xl = lax.dot_general(x_ref[...], wl_ref[...], (((1,), (1,)), ((), ())),
                             preferred_element_type=jnp.float32)
        y_ref[...] = jnp.dot(xl, b_ref[...], preferred_element_type=jnp.float32)
        at_ref[...] = lax.dot_general(
            aw_ref[...], xl, (((0,), (1,)), ((), ())),
            preferred_element_type=jnp.float32)

    return pl.pallas_call(
        body,
        out_shape=(
            jax.ShapeDtypeStruct((N_NODES, 256), jnp.float32),
            jax.ShapeDtypeStruct((16, N_NODES), jnp.float32),
        ),
    )(x, W_lin, B, attW)


def _lrelu(v):
    return jnp.where(v >= 0.0, v, NEG * v)


def _pass_a(AT, row, col):
    mesh = plsc.VectorSubcoreMesh(core_axis_name="c", subcore_axis_name="s")

    @functools.partial(
        pl.kernel, mesh=mesh,
        compiler_params=pltpu.CompilerParams(needs_layout_passes=False),
        out_type=(
            jax.ShapeDtypeStruct((HEADS * N_EDGES,), jnp.float32),
            jax.ShapeDtypeStruct((NW * 16,), jnp.float32),
        ),
        scratch_types=[
            pltpu.VMEM((N_NODES,), jnp.float32),
            pltpu.VMEM((N_NODES,), jnp.float32),
            pltpu.VMEM((2 * A_CHUNK,), jnp.int32),
            pltpu.VMEM((2 * A_CHUNK,), jnp.int32),
            pltpu.VMEM((2 * A_CHUNK,), jnp.float32),
            pltpu.VMEM((16,), jnp.float32),
            pltpu.SemaphoreType.DMA((2,)),
            pltpu.SemaphoreType.DMA((2,)),
        ],
    )
    def k(at_hbm, row_hbm, col_hbm, p_hbm, part_hbm, al_v, ar_v, ir_v, ic_v,
          po_v, sp_v, sem_i, sem_w):
        wid = lax.axis_index("s") * 2 + lax.axis_index("c")
        h = wid // 4
        q = wid % 4
        base = q * A_EPT

        pltpu.sync_copy(at_hbm.at[pl.ds(h * N_NODES, N_NODES)], al_v)
        pltpu.sync_copy(at_hbm.at[pl.ds((h + HEADS) * N_NODES, N_NODES)], ar_v)

        def mx(i, carry):
            ml, mr = carry
            ml = jnp.maximum(ml, al_v[pl.ds(i * 16, 16)])
            mr = jnp.maximum(mr, ar_v[pl.ds(i * 16, 16)])
            return ml, mr
        neg = jnp.full((16,), -3e38, jnp.float32)
        ml, mr = lax.fori_loop(0, N_NODES // 16, mx, (neg, neg))

        lanes = lax.iota(jnp.int32, 16)

        def butterfly(v, op):
            for k in (1, 2, 4, 8):
                sp_v[...] = v
                v = op(v, plsc.load_gather(sp_v, [lanes ^ k]))
            return v

        shift = _lrelu(butterfly(ml, jnp.maximum) + butterfly(mr, jnp.maximum))

        ncnk = A_EPT // A_CHUNK

        def issue_idx(cidx):
            ebase = base + cidx * A_CHUNK
            par = cidx & 1
            pltpu.async_copy(row_hbm.at[pl.ds(ebase, A_CHUNK)],
                             ir_v.at[pl.ds(par * A_CHUNK, A_CHUNK)],
                             sem_i.at[par])
            pltpu.async_copy(col_hbm.at[pl.ds(ebase, A_CHUNK)],
                             ic_v.at[pl.ds(par * A_CHUNK, A_CHUNK)],
                             sem_i.at[par])

        issue_idx(0)

        def chunk(cidx, acc):
            par = cidx & 1

            @pl.when(cidx < ncnk - 1)
            def _():
                issue_idx(cidx + 1)

            pltpu.make_async_copy(row_hbm.at[pl.ds(base, A_CHUNK)],
                                  ir_v.at[pl.ds(par * A_CHUNK, A_CHUNK)],
                                  sem_i.at[par]).wait()
            pltpu.make_async_copy(col_hbm.at[pl.ds(base, A_CHUNK)],
                                  ic_v.at[pl.ds(par * A_CHUNK, A_CHUNK)],
                                  sem_i.at[par]).wait()

            @pl.when(cidx >= 2)
            def _():
                pltpu.make_async_copy(
                    po_v.at[pl.ds(par * A_CHUNK, A_CHUNK)],
                    p_hbm.at[pl.ds(h * N_EDGES + base, A_CHUNK)],
                    sem_w.at[par]).wait()

            def grp(g, acc):
                rv = ir_v[pl.ds(par * A_CHUNK + g * 16, 16)]
                cv = ic_v[pl.ds(par * A_CHUNK + g * 16, 16)]
                s = plsc.load_gather(al_v, [rv]) + plsc.load_gather(ar_v, [cv])
                p = jnp.exp(_lrelu(s) - shift)
                po_v[pl.ds(par * A_CHUNK + g * 16, 16)] = p
                return acc + p

            acc = lax.fori_loop(0, A_CHUNK // 16, grp, acc)
            ebase = base + cidx * A_CHUNK
            pltpu.async_copy(po_v.at[pl.ds(par * A_CHUNK, A_CHUNK)],
                             p_hbm.at[pl.ds(h * N_EDGES + ebase, A_CHUNK)],
                             sem_w.at[par])
            return acc

        acc = lax.fori_loop(0, ncnk, chunk, jnp.zeros((16,), jnp.float32))
        for par in range(2):
            pltpu.make_async_copy(
                po_v.at[pl.ds(par * A_CHUNK, A_CHUNK)],
                p_hbm.at[pl.ds(h * N_EDGES + base, A_CHUNK)],
                sem_w.at[par]).wait()
        total = butterfly(acc, jnp.add)
        sp_v[...] = total
        pltpu.sync_copy(sp_v, part_hbm.at[pl.ds(wid * 16, 16)])

    return k(AT, row, col)


def _pass_b(Y, col, p, part, b_out):
    mesh = plsc.VectorSubcoreMesh(core_axis_name="c", subcore_axis_name="s")

    SUP = 2000
    NSUP = B_EPT // SUP
    CPS = SUP // B_CHUNK

    @functools.partial(
        pl.kernel, mesh=mesh,
        compiler_params=pltpu.CompilerParams(needs_layout_passes=False),
        out_type=jax.ShapeDtypeStruct((N_EDGES, OUT_CH), jnp.float32),
        scratch_types=[
            pltpu.VMEM((B_EPT,), jnp.int32),
            pltpu.VMEM((2 * B_CHUNK, 256), jnp.float32),
            pltpu.VMEM((HEADS * SUP,), jnp.float32),
            pltpu.VMEM((2 * B_CHUNK, OUT_CH), jnp.float32),
            pltpu.VMEM((NW * 16,), jnp.float32),
            pltpu.VMEM((OUT_CH,), jnp.float32),
            pltpu.SemaphoreType.DMA((2,)),
            pltpu.SemaphoreType.DMA((2,)),
            pltpu.SemaphoreType.DMA,
        ],
    )
    def k(y_hbm, col_hbm, p_hbm, part_hbm, b_hbm, out_hbm, ic_v, y_v, p_v,
          o_v, part_v, b_v, sem_g, sem_w, sem_p):
        wid = lax.axis_index("s") * 2 + lax.axis_index("c")
        base = wid * B_EPT

        pltpu.sync_copy(col_hbm.at[pl.ds(base, B_EPT)], ic_v)
        pltpu.sync_copy(part_hbm, part_v)
        pltpu.sync_copy(b_hbm, b_v)
        inv = []
        for h in range(HEADS):
            d = (part_v[pl.ds((4 * h) * 16, 16)]
                 + part_v[pl.ds((4 * h + 1) * 16, 16)]
                 + part_v[pl.ds((4 * h + 2) * 16, 16)]
                 + part_v[pl.ds((4 * h + 3) * 16, 16)])
            inv.append(1.0 / d)
        blo = b_v[pl.ds(0, 16)]
        bhi = b_v[pl.ds(16, 16)]
        off8 = (lax.iota(jnp.int32, 16) & 7) * SUP
        one16 = jnp.full((16,), 1, jnp.int32)

        def issue_gather(sup, c):
            par = c & 1
            pltpu.async_copy(
                y_hbm.at[ic_v.at[pl.ds(sup * SUP + c * B_CHUNK, B_CHUNK)]],
                y_v.at[pl.ds(par * B_CHUNK, B_CHUNK)], sem_g.at[par])

        def super_chunk(sup, _):
            hs = []
            for h in range(HEADS):
                hs.append(pltpu.async_copy(
                    p_hbm.at[pl.ds(h * N_EDGES + base + sup * SUP, SUP)],
                    p_v.at[pl.ds(h * SUP, SUP)], sem_p))
            for c in hs:
                c.wait()
            for h in range(HEADS):
                iv = inv[h]

                def nrm(g, _):
                    ix = h * SUP + g * 16
                    p_v[pl.ds(ix, 16)] = p_v[pl.ds(ix, 16)] * iv
                    return 0
                lax.fori_loop(0, SUP // 16, nrm, 0)

            issue_gather(sup, 0)

            def chunk(c, _):
                par = c & 1

                @pl.when(c < CPS - 1)
                def _():
                    issue_gather(sup, c + 1)

                pltpu.make_async_copy(
                    y_hbm.at[ic_v.at[pl.ds(0, B_CHUNK)]],
                    y_v.at[pl.ds(par * B_CHUNK, B_CHUNK)],
                    sem_g.at[par]).wait()

                @pl.when(c >= 2)
                def _():
                    pltpu.make_async_copy(
                        o_v.at[pl.ds(par * B_CHUNK, B_CHUNK)],
                        out_hbm.at[pl.ds(base, B_CHUNK)],
                        sem_w.at[par]).wait()

                def edge(e, _):
                    a8 = plsc.load_gather(p_v, [off8 + one16 * (c * B_CHUNK + e)])
                    acc_lo = blo
                    acc_hi = bhi
                    for h in range(HEADS):
                        a = a8[h]
                        acc_lo = acc_lo + a * y_v[par * B_CHUNK + e, pl.ds(h * 32, 16)]
                        acc_hi = acc_hi + a * y_v[par * B_CHUNK + e, pl.ds(h * 32 + 16, 16)]
                    o_v[par * B_CHUNK + e, pl.ds(0, 16)] = acc_lo
                    o_v[par * B_CHUNK + e, pl.ds(16, 16)] = acc_hi
                    return 0

                lax.fori_loop(0, B_CHUNK, edge, 0)
                ebase = base + sup * SUP + c * B_CHUNK
                pltpu.async_copy(o_v.at[pl.ds(par * B_CHUNK, B_CHUNK)],
                                 out_hbm.at[pl.ds(ebase, B_CHUNK)],
                                 sem_w.at[par])
                return 0

            lax.fori_loop(0, CPS, chunk, 0)
            for par in range(2):
                pltpu.make_async_copy(
                    o_v.at[pl.ds(par * B_CHUNK, B_CHUNK)],
                    out_hbm.at[pl.ds(base, B_CHUNK)],
                    sem_w.at[par]).wait()
            return 0

        lax.fori_loop(0, NSUP, super_chunk, 0)

    return k(Y, col, p, part, b_out)


def kernel(x, edge_index, W_lin, att, W_out, b_out):
    row = edge_index[0].astype(jnp.int32)
    col = edge_index[1].astype(jnp.int32)

    eye8 = jnp.eye(HEADS, dtype=jnp.float32)
    W_t = W_out.reshape(OUT_CH, HEADS, OUT_CH).transpose(1, 2, 0)
    B = (eye8[:, None, :, None] * W_t[:, :, None, :]).reshape(256, 256)
    att_l = att[0, :, :OUT_CH]
    att_r = att[0, :, OUT_CH:]
    attW_l = (eye8[:, None, :] * att_l[:, :, None]).reshape(256, HEADS)
    attW_r = (eye8[:, None, :] * att_r[:, :, None]).reshape(256, HEADS)
    attW = jnp.concatenate([attW_l, attW_r], axis=1)

    Y, AT = _tc_dense(x, W_lin, B, attW)
    p, part = _pass_a(AT.reshape(-1), row, col)
    out = _pass_b(Y, col, p, part, b_out)
    return out
